# baseline (speedup 1.0000x reference)
import functools

import jax
import jax.numpy as jnp
from jax import lax
from jax.experimental import pallas as pl
from jax.experimental.pallas import tpu as pltpu
from jax.experimental.pallas import tpu_sc as plsc

_INFO = plsc.get_sparse_core_info()
_NC, _NS, _L = _INFO.num_cores, _INFO.num_subcores, _INFO.num_lanes
_NW = _NC * _NS


@functools.partial(jax.jit, static_argnames=("rows", "cols", "nidx"))
def _gather_elements(in_flat, idx_flat, *, rows, cols, nidx):
    rows_per_w = rows // _NW
    chunk = 64
    n_chunks = rows_per_w // chunk
    pair_vecs = (2 * nidx) // _L

    mesh = plsc.VectorSubcoreMesh(core_axis_name="c", subcore_axis_name="s")

    @functools.partial(
        pl.kernel,
        out_type=jax.ShapeDtypeStruct((rows * nidx,), jnp.float32),
        mesh=mesh,
        compiler_params=pltpu.CompilerParams(needs_layout_passes=False),
        scratch_types=[
            pltpu.VMEM((chunk * cols,), jnp.float32),
            pltpu.VMEM((chunk * nidx,), jnp.int32),
            pltpu.VMEM((chunk * nidx,), jnp.float32),
        ],
    )
    def k(in_hbm, idx_hbm, out_hbm, in_v, idx_v, out_v):
        wid = lax.axis_index("s") * _NC + lax.axis_index("c")
        w_row0 = wid * rows_per_w
        iota = lax.broadcasted_iota(jnp.int32, (_L,), 0)

        def chunk_body(c, _):
            row0 = w_row0 + c * chunk
            pltpu.sync_copy(in_hbm.at[pl.ds(row0 * cols, chunk * cols)], in_v)
            pltpu.sync_copy(idx_hbm.at[pl.ds(row0 * nidx, chunk * nidx)], idx_v)

            def pair_body(p, _):
                fbase = p * (2 * cols)
                for v in range(pair_vecs):
                    base = p * (2 * nidx) + v * _L
                    roff = jnp.where((v * _L + iota) >= nidx, cols, 0)
                    colv = idx_v[pl.ds(base, _L)]
                    out_v[pl.ds(base, _L)] = plsc.load_gather(
                        in_v, [colv + fbase + roff])
                return 0

            lax.fori_loop(0, chunk // 2, pair_body, 0)
            pltpu.sync_copy(out_v, out_hbm.at[pl.ds(row0 * nidx, chunk * nidx)])
            return 0

        lax.fori_loop(0, n_chunks, chunk_body, 0)

    return k(in_flat, idx_flat)


def kernel(input_tensor, indices):
    rows, cols = input_tensor.shape
    nidx = indices.shape[1]
    in_flat = input_tensor.reshape(-1)
    idx_flat = indices.astype(jnp.int32).reshape(-1)
    out = _gather_elements(in_flat, idx_flat, rows=rows, cols=cols, nidx=nidx)
    return out.reshape(rows, nidx)


# speedup vs baseline: 1.1203x; 1.1203x over previous
import functools

import jax
import jax.numpy as jnp
from jax import lax
from jax.experimental import pallas as pl
from jax.experimental.pallas import tpu as pltpu
from jax.experimental.pallas import tpu_sc as plsc

_INFO = plsc.get_sparse_core_info()
_NC, _NS, _L = _INFO.num_cores, _INFO.num_subcores, _INFO.num_lanes
_NW = _NC * _NS


@functools.partial(jax.jit, static_argnames=("rows", "cols", "nidx"))
def _gather_elements(in_flat, idx_flat, *, rows, cols, nidx):
    rows_per_w = rows // _NW
    chunk = 32
    n_chunks = rows_per_w // chunk
    pair_vecs = (2 * nidx) // _L

    mesh = plsc.VectorSubcoreMesh(core_axis_name="c", subcore_axis_name="s")

    @functools.partial(
        pl.kernel,
        out_type=jax.ShapeDtypeStruct((rows * nidx,), jnp.float32),
        mesh=mesh,
        compiler_params=pltpu.CompilerParams(needs_layout_passes=False),
        scratch_types=[
            [pltpu.VMEM((chunk * cols,), jnp.float32) for _ in range(2)],
            [pltpu.VMEM((chunk * nidx,), jnp.int32) for _ in range(2)],
            [pltpu.VMEM((chunk * nidx,), jnp.float32) for _ in range(2)],
            [pltpu.SemaphoreType.DMA for _ in range(6)],
        ],
    )
    def k(in_hbm, idx_hbm, out_hbm, in_v, idx_v, out_v, sems):
        wid = lax.axis_index("s") * _NC + lax.axis_index("c")
        w_row0 = wid * rows_per_w
        iota = lax.broadcasted_iota(jnp.int32, (_L,), 0)

        def start_in(c):
            b = c % 2
            row0 = w_row0 + c * chunk
            d1 = pltpu.async_copy(
                in_hbm.at[pl.ds(row0 * cols, chunk * cols)], in_v[b], sems[b])
            d2 = pltpu.async_copy(
                idx_hbm.at[pl.ds(row0 * nidx, chunk * nidx)], idx_v[b],
                sems[2 + b])
            return d1, d2

        def start_out(c):
            b = c % 2
            row0 = w_row0 + c * chunk
            return pltpu.async_copy(
                out_v[b], out_hbm.at[pl.ds(row0 * nidx, chunk * nidx)],
                sems[4 + b])

        def compute(c):
            b = c % 2
            iv, xv, ov = in_v[b], idx_v[b], out_v[b]

            def pair_body(p, _):
                fbase = p * (2 * cols)
                for v in range(pair_vecs):
                    base = p * (2 * nidx) + v * _L
                    roff = jnp.where((v * _L + iota) >= nidx, cols, 0)
                    colv = xv[pl.ds(base, _L)]
                    ov[pl.ds(base, _L)] = plsc.load_gather(
                        iv, [colv + fbase + roff])
                return 0

            lax.fori_loop(0, chunk // 2, pair_body, 0)

        d_in = {0: start_in(0)}
        d_out = {}
        for c in range(n_chunks):
            if c + 1 < n_chunks:
                d_in[c + 1] = start_in(c + 1)
            for d in d_in.pop(c):
                d.wait()
            if c >= 2:
                d_out.pop(c - 2).wait()
            compute(c)
            d_out[c] = start_out(c)
        for c in sorted(d_out):
            d_out.pop(c).wait()

    return k(in_flat, idx_flat)


def kernel(input_tensor, indices):
    rows, cols = input_tensor.shape
    nidx = indices.shape[1]
    in_flat = input_tensor.reshape(-1)
    idx_flat = indices.astype(jnp.int32).reshape(-1)
    out = _gather_elements(in_flat, idx_flat, rows=rows, cols=cols, nidx=nidx)
    return out.reshape(rows, nidx)


# speedup vs baseline: 1.2498x; 1.1156x over previous
import functools

import jax
import jax.numpy as jnp
from jax import lax
from jax.experimental import pallas as pl
from jax.experimental.pallas import tpu as pltpu
from jax.experimental.pallas import tpu_sc as plsc

_INFO = plsc.get_sparse_core_info()
_NC, _NS, _L = _INFO.num_cores, _INFO.num_subcores, _INFO.num_lanes
_NW = _NC * _NS


@functools.partial(jax.jit, static_argnames=("rows", "cols", "nidx"))
def _gather_elements(in_flat, idx_flat, *, rows, cols, nidx):
    rows_per_w = rows // _NW
    chunk = 32
    n_chunks = rows_per_w // chunk
    pair_vecs = (2 * nidx) // _L

    mesh = plsc.VectorSubcoreMesh(core_axis_name="c", subcore_axis_name="s")

    @functools.partial(
        pl.kernel,
        out_type=jax.ShapeDtypeStruct((rows * nidx,), jnp.float32),
        mesh=mesh,
        compiler_params=pltpu.CompilerParams(needs_layout_passes=False),
        scratch_types=[
            [pltpu.VMEM((chunk * cols,), jnp.float32) for _ in range(2)],
            [pltpu.VMEM((chunk * nidx,), jnp.int32) for _ in range(2)],
            [pltpu.VMEM((chunk * nidx,), jnp.float32) for _ in range(2)],
            [pltpu.SemaphoreType.DMA for _ in range(6)],
        ],
    )
    def k(in_hbm, idx_hbm, out_hbm, in_v, idx_v, out_v, sems):
        wid = lax.axis_index("s") * _NC + lax.axis_index("c")
        w_row0 = wid * rows_per_w
        iota = lax.broadcasted_iota(jnp.int32, (_L,), 0)

        def start_in(c, b):
            row0 = w_row0 + c * chunk
            pltpu.async_copy(
                in_hbm.at[pl.ds(row0 * cols, chunk * cols)], in_v[b], sems[b])
            pltpu.async_copy(
                idx_hbm.at[pl.ds(row0 * nidx, chunk * nidx)], idx_v[b],
                sems[2 + b])

        def wait_in(b):
            pltpu.make_async_copy(
                in_hbm.at[pl.ds(0, chunk * cols)], in_v[b], sems[b]).wait()
            pltpu.make_async_copy(
                idx_hbm.at[pl.ds(0, chunk * nidx)], idx_v[b],
                sems[2 + b]).wait()

        def start_out(c, b):
            row0 = w_row0 + c * chunk
            pltpu.async_copy(
                out_v[b], out_hbm.at[pl.ds(row0 * nidx, chunk * nidx)],
                sems[4 + b])

        def wait_out(b):
            pltpu.make_async_copy(
                out_v[b], out_hbm.at[pl.ds(0, chunk * nidx)],
                sems[4 + b]).wait()

        def compute(b):
            iv, xv, ov = in_v[b], idx_v[b], out_v[b]

            @plsc.parallel_loop(0, chunk // 2, unroll=1)
            def pair_body(p):
                fbase = p * (2 * cols)
                for v in range(pair_vecs):
                    base = p * (2 * nidx) + v * _L
                    roff = jnp.where((v * _L + iota) >= nidx, cols, 0)
                    colv = xv[pl.ds(base, _L)]
                    ov[pl.ds(base, _L)] = plsc.load_gather(
                        iv, [colv + fbase + roff])

        start_in(0, 0)
        start_in(1, 1)
        for b in (0, 1):
            wait_in(b)
            compute(b)
            start_out(b, b)
            start_in(b + 2, b)

        def super_body(g, _):
            c0 = 2 * g
            for b in (0, 1):
                wait_in(b)
                wait_out(b)
                compute(b)
                start_out(c0 + b, b)
                start_in(c0 + b + 2, b)
            return 0

        lax.fori_loop(1, n_chunks // 2 - 1, super_body, 0)

        for b in (0, 1):
            wait_in(b)
            wait_out(b)
            compute(b)
            start_out(n_chunks - 2 + b, b)
        wait_out(0)
        wait_out(1)

    return k(in_flat, idx_flat)


def kernel(input_tensor, indices):
    rows, cols = input_tensor.shape
    nidx = indices.shape[1]
    in_flat = input_tensor.reshape(-1)
    idx_flat = indices.astype(jnp.int32).reshape(-1)
    out = _gather_elements(in_flat, idx_flat, rows=rows, cols=cols, nidx=nidx)
    return out.reshape(rows, nidx)
